# baseline (speedup 1.0000x reference)
import numpy as np
import jax
import jax.numpy as jnp
from jax import lax
from jax.experimental import pallas as pl
from jax.experimental.pallas import tpu as pltpu
from jax.experimental.pallas import tpu_sc as plsc

B = 4096
ND = 13
NF = 26
VOCAB = 1000
EMB = 32
SLOTS = NF + 2
DP = SLOTS * EMB
D = ND + NF * EMB
H1 = 1024
H2 = 1024
OUT_DIM = 256
NCROSS = 4
BM = 512
NDP = 39


def _gather_x(table_aug, idx3, nw, nc, nchunk, rpw):
    mesh = plsc.VectorSubcoreMesh(core_axis_name="c", subcore_axis_name="s")

    def body(table_hbm, idx_hbm, x_hbm, idx_v, rows_v, sem):
        w = lax.axis_index("s") * nc + lax.axis_index("c")
        pltpu.sync_copy(idx_hbm.at[w], idx_v)

        def step(i, carry):
            cps = [
                pltpu.async_copy(
                    table_hbm.at[idx_v.at[i * 4 + u]],
                    rows_v.at[pl.ds((i * 4 + u) * 128, 128), :], sem,
                )
                for u in range(4)
            ]
            for cp in cps:
                cp.wait()
            return carry

        lax.fori_loop(0, nchunk // 4, step, 0)
        pltpu.sync_copy(rows_v, x_hbm.at[pl.ds(w * rpw, rpw), :])

    k = pl.kernel(
        body,
        out_type=jax.ShapeDtypeStruct((B * SLOTS, EMB), jnp.float32),
        mesh=mesh,
        compiler_params=pltpu.CompilerParams(use_tc_tiling_on_sc=False),
        scratch_types=[
            pltpu.VMEM((nchunk, 128), jnp.int32),
            pltpu.VMEM((rpw, EMB), jnp.float32),
            pltpu.SemaphoreType.DMA,
        ],
    )
    return k(table_aug, idx3)


def _dcn_tc(x4d, xraw, cwe, cwd, betag, w1e, w1d, b1, W2, b2, W3, b3, woh, bo):
    nck = DP // 128

    def body(x_ref, xd_ref, cwe_ref, cwd_ref, bg_ref, w1e_ref, w1d_ref,
             b1_ref, w2_ref, b2_ref, w3_ref, b3_ref, woh_ref, bo_ref, out_ref):
        xd = xd_ref[...].astype(jnp.bfloat16)
        xcs = [
            x_ref[:, c].reshape(BM, 128).astype(jnp.bfloat16) for c in range(nck)
        ]
        P = jnp.dot(xd, cwd_ref[...], preferred_element_type=jnp.float32)
        for c in range(nck):
            P = P + jnp.dot(xcs[c], cwe_ref[c],
                            preferred_element_type=jnp.float32)
        c_l = jnp.ones((BM, 1), jnp.float32)
        for l in range(NCROSS):
            c_l = c_l + c_l * P[:, l:l + 1] + bg_ref[0, l]
        gamma = bg_ref[0, NCROSS]
        h = jnp.dot(xd, w1d_ref[...], preferred_element_type=jnp.float32)
        for c in range(nck):
            h = h + jnp.dot(xcs[c], w1e_ref[c],
                            preferred_element_type=jnp.float32)
        h = jnp.maximum(h + b1_ref[...], 0.0).astype(jnp.bfloat16)
        h = jnp.maximum(
            jnp.dot(h, w2_ref[...], preferred_element_type=jnp.float32)
            + b2_ref[...], 0.0).astype(jnp.bfloat16)
        h = jnp.maximum(
            jnp.dot(h, w3_ref[...], preferred_element_type=jnp.float32)
            + b3_ref[...], 0.0).astype(jnp.bfloat16)
        logit = (c_l * P[:, NCROSS:NCROSS + 1] + gamma
                 + jnp.dot(h, woh_ref[...], preferred_element_type=jnp.float32)
                 + bo_ref[0, 0])
        out_ref[...] = 1.0 / (1.0 + jnp.exp(-logit))

    def wspec(shape):
        return pl.BlockSpec(shape, lambda i: (0,) * len(shape))

    return pl.pallas_call(
        body,
        grid=(B // BM,),
        in_specs=[
            pl.BlockSpec((BM // 8, nck, 8, 128), lambda i: (i, 0, 0, 0)),
            pl.BlockSpec((BM, NDP), lambda i: (i, 0)),
            wspec((nck, 128, NCROSS + 1)),
            wspec((NDP, NCROSS + 1)),
            wspec((1, 8)),
            wspec((nck, 128, H1)),
            wspec((NDP, H1)),
            wspec((1, H1)),
            wspec((H1, H2)),
            wspec((1, H2)),
            wspec((H2, OUT_DIM)),
            wspec((1, OUT_DIM)),
            wspec((OUT_DIM, 1)),
            wspec((1, 1)),
        ],
        out_specs=pl.BlockSpec((BM, 1), lambda i: (i, 0)),
        out_shape=jax.ShapeDtypeStruct((B, 1), jnp.float32),
        compiler_params=pltpu.CompilerParams(
            dimension_semantics=("arbitrary",)),
    )(x4d, xraw, cwe, cwd, betag, w1e, w1d, b1, W2, b2, W3, b3, woh, bo)


def kernel(inputs, embed_tables, cross_w, cross_b, W1, b1, W2, b2, W3, b3, Wo, bo):
    bf16 = jnp.bfloat16
    f32 = jnp.float32
    info = plsc.get_sparse_core_info()
    nc, ns = info.num_cores, info.num_subcores
    nw = nc * ns
    rpw = (B * SLOTS) // nw
    nchunk = rpw // 128
    bpw = B // nw

    sp_idx = inputs[:, ND:].astype(jnp.int32)

    table_aug = jnp.concatenate(
        [embed_tables.reshape(NF * VOCAB, EMB),
         jnp.zeros((1, EMB), f32)], axis=0)

    zero_idx = jnp.full((B, SLOTS - NF), NF * VOCAB, jnp.int32)
    emb_idx = sp_idx + (jnp.arange(NF, dtype=jnp.int32) * VOCAB)[None, :]
    idx_all = jnp.concatenate([emb_idx, zero_idx], axis=1)
    idx3 = (idx_all.reshape(nw, bpw // 8, 8, SLOTS // 4, 4)
            .transpose(0, 1, 3, 2, 4).reshape(nw, nchunk, 128))

    x_flat = _gather_x(table_aug, idx3, nw, nc, nchunk, rpw)
    x4d = x_flat.reshape(B // 8, DP // 128, 8, 128)

    def embw(m):
        k = m.shape[0]
        return (jnp.concatenate([m, jnp.zeros((k, DP - NF * EMB), f32)], axis=1)
                .T.reshape(DP // 128, 128, k).astype(bf16))

    def densew(m):
        k = m.shape[0]
        return (jnp.concatenate([m, jnp.zeros((k, NDP - ND), f32)], axis=1)
                .T.astype(bf16))

    cw5 = jnp.concatenate([cross_w, Wo[:D].reshape(1, D)], axis=0)
    cwe = embw(cw5[:, ND:])
    cwd = densew(cw5[:, :ND])
    w1e = embw(W1[ND:].T)
    w1d = densew(W1[:ND].T)

    bcum = jnp.cumsum(cross_b, axis=0)
    beta = jnp.concatenate([
        jnp.zeros((1,), f32),
        jnp.sum(bcum[:NCROSS - 1] * cross_w[1:], axis=1)])
    gamma = jnp.sum(bcum[NCROSS - 1] * Wo[:D, 0])
    betag = jnp.concatenate(
        [beta, gamma[None], jnp.zeros((3,), f32)]).reshape(1, 8)

    woh = Wo[D:].astype(bf16)

    return _dcn_tc(
        x4d, inputs, cwe, cwd, betag, w1e, w1d,
        b1.reshape(1, H1), W2.astype(bf16), b2.reshape(1, H2),
        W3.astype(bf16), b3.reshape(1, OUT_DIM), woh, bo.reshape(1, 1))


# speedup vs baseline: 2.1024x; 2.1024x over previous
import jax
import jax.numpy as jnp
from jax import lax
from jax.experimental import pallas as pl
from jax.experimental.pallas import tpu as pltpu
from jax.experimental.pallas import tpu_sc as plsc

B = 4096
ND = 13
NF = 26
VOCAB = 1000
EMB = 32
SLOTS = NF + 2
DP = SLOTS * EMB
D = ND + NF * EMB
H1 = 1024
H2 = 1024
OUT_DIM = 256
NCROSS = 4
BM = 512


def _gather_x(table_aug, idx3, nw, nc, nchunk, rpw):
    mesh = plsc.VectorSubcoreMesh(core_axis_name="c", subcore_axis_name="s")

    def body(table_hbm, idx_hbm, x_hbm, idx_v, rows_v, sem):
        w = lax.axis_index("s") * nc + lax.axis_index("c")
        pltpu.sync_copy(idx_hbm.at[w], idx_v)

        def step(i, carry):
            cps = [
                pltpu.async_copy(
                    table_hbm.at[idx_v.at[i * 4 + u]],
                    rows_v.at[pl.ds((i * 4 + u) * 128, 128), :], sem,
                )
                for u in range(4)
            ]
            for cp in cps:
                cp.wait()
            return carry

        lax.fori_loop(0, nchunk // 4, step, 0)
        pltpu.sync_copy(rows_v, x_hbm.at[pl.ds(w * rpw, rpw), :])

    k = pl.kernel(
        body,
        out_type=jax.ShapeDtypeStruct((B * SLOTS, EMB), jnp.float32),
        mesh=mesh,
        compiler_params=pltpu.CompilerParams(use_tc_tiling_on_sc=False),
        scratch_types=[
            pltpu.VMEM((nchunk, 128), jnp.int32),
            pltpu.VMEM((rpw, EMB), jnp.float32),
            pltpu.SemaphoreType.DMA,
        ],
    )
    return k(table_aug, idx3)


def _dcn_tc(x4d, cw_all, betag, W1p, b1, W2, b2, W3, b3, woh, bo):
    nck = DP // 128

    def body(x_ref, cwall_ref, bg_ref, w1_ref, b1_ref,
             w2_ref, b2_ref, w3_ref, b3_ref, woh_ref, bo_ref, out_ref):
        x = jnp.concatenate(
            [x_ref[:, c].reshape(BM, 128) for c in range(nck)],
            axis=1).astype(jnp.bfloat16)
        P = jnp.dot(x, cwall_ref[...], preferred_element_type=jnp.float32)
        c_l = jnp.ones((BM, 1), jnp.float32)
        for l in range(NCROSS):
            c_l = c_l + c_l * P[:, l:l + 1] + bg_ref[0, l]
        gamma = bg_ref[0, NCROSS]
        h = jnp.maximum(
            jnp.dot(x, w1_ref[...], preferred_element_type=jnp.float32)
            + b1_ref[...], 0.0).astype(jnp.bfloat16)
        h = jnp.maximum(
            jnp.dot(h, w2_ref[...], preferred_element_type=jnp.float32)
            + b2_ref[...], 0.0).astype(jnp.bfloat16)
        h = jnp.maximum(
            jnp.dot(h, w3_ref[...], preferred_element_type=jnp.float32)
            + b3_ref[...], 0.0).astype(jnp.bfloat16)
        logit = (c_l * P[:, NCROSS:NCROSS + 1] + gamma
                 + jnp.dot(h, woh_ref[...], preferred_element_type=jnp.float32)
                 + bo_ref[0, 0])
        out_ref[...] = 1.0 / (1.0 + jnp.exp(-logit))

    def wspec(shape):
        return pl.BlockSpec(shape, lambda i: (0,) * len(shape))

    return pl.pallas_call(
        body,
        grid=(B // BM,),
        in_specs=[
            pl.BlockSpec((BM // 8, nck, 8, 128), lambda i: (i, 0, 0, 0)),
            wspec((DP, NCROSS + 1)),
            wspec((1, 8)),
            wspec((DP, H1)),
            wspec((1, H1)),
            wspec((H1, H2)),
            wspec((1, H2)),
            wspec((H2, OUT_DIM)),
            wspec((1, OUT_DIM)),
            wspec((OUT_DIM, 1)),
            wspec((1, 1)),
        ],
        out_specs=pl.BlockSpec((BM, 1), lambda i: (i, 0)),
        out_shape=jax.ShapeDtypeStruct((B, 1), jnp.float32),
        compiler_params=pltpu.CompilerParams(
            dimension_semantics=("arbitrary",)),
    )(x4d, cw_all, betag, W1p, b1, W2, b2, W3, b3, woh, bo)


def kernel(inputs, embed_tables, cross_w, cross_b, W1, b1, W2, b2, W3, b3, Wo, bo):
    bf16 = jnp.bfloat16
    f32 = jnp.float32
    info = plsc.get_sparse_core_info()
    nc, ns = info.num_cores, info.num_subcores
    nw = nc * ns
    rpw = (B * SLOTS) // nw
    nchunk = rpw // 128
    bpw = B // nw

    sp_idx = inputs[:, ND:].astype(jnp.int32)

    dense_rows = jnp.pad(inputs[:, :ND], ((0, 0), (0, EMB - ND)))
    table_aug = jnp.concatenate(
        [embed_tables.reshape(NF * VOCAB, EMB), dense_rows], axis=0)

    emb_idx = sp_idx + (jnp.arange(NF, dtype=jnp.int32) * VOCAB)[None, :]
    dense_idx = NF * VOCAB + jnp.arange(B, dtype=jnp.int32)[:, None]
    idx_all = jnp.concatenate(
        [emb_idx, dense_idx, dense_idx], axis=1)
    idx3 = (idx_all.reshape(nw, bpw // 8, 8, SLOTS // 4, 4)
            .transpose(0, 1, 3, 2, 4).reshape(nw, nchunk, 128))

    x_flat = _gather_x(table_aug, idx3, nw, nc, nchunk, rpw)
    x4d = x_flat.reshape(B // 8, DP // 128, 8, 128)

    def padw(m):
        k = m.shape[0]
        return (jnp.concatenate(
            [m[:, ND:], m[:, :ND], jnp.zeros((k, DP - D), f32)], axis=1)
            .T.astype(bf16))

    cw_all = padw(jnp.concatenate([cross_w, Wo[:D].reshape(1, D)], axis=0))
    W1p = padw(W1.T)

    bcum = jnp.cumsum(cross_b, axis=0)
    beta = jnp.concatenate([
        jnp.zeros((1,), f32),
        jnp.sum(bcum[:NCROSS - 1] * cross_w[1:], axis=1)])
    gamma = jnp.sum(bcum[NCROSS - 1] * Wo[:D, 0])
    betag = jnp.concatenate(
        [beta, gamma[None], jnp.zeros((3,), f32)]).reshape(1, 8)

    woh = Wo[D:].astype(bf16)

    return _dcn_tc(
        x4d, cw_all, betag, W1p,
        b1.reshape(1, H1), W2.astype(bf16), b2.reshape(1, H2),
        W3.astype(bf16), b3.reshape(1, OUT_DIM), woh, bo.reshape(1, 1))


# speedup vs baseline: 2.2761x; 1.0826x over previous
import jax
import jax.numpy as jnp
from jax import lax
from jax.experimental import pallas as pl
from jax.experimental.pallas import tpu as pltpu
from jax.experimental.pallas import tpu_sc as plsc

B = 4096
ND = 13
NF = 26
VOCAB = 1000
EMB = 32
SLOTS = NF + 2
DP = SLOTS * EMB
DE = NF * EMB
D = ND + DE
H1 = 1024
H2 = 1024
OUT_DIM = 256
NCROSS = 4
BM = 512
NDP = 39


def _gather_x(table, idx3, nw, nc, nchunk, rpw):
    mesh = plsc.VectorSubcoreMesh(core_axis_name="c", subcore_axis_name="s")

    def body(table_hbm, idx_hbm, x_hbm, idx_v, rows_v, sem):
        w = lax.axis_index("s") * nc + lax.axis_index("c")
        pltpu.sync_copy(idx_hbm.at[w], idx_v)

        def step(i, carry):
            cps = [
                pltpu.async_copy(
                    table_hbm.at[idx_v.at[i * 4 + u]],
                    rows_v.at[pl.ds((i * 4 + u) * 128, 128), :], sem,
                )
                for u in range(4)
            ]
            for cp in cps:
                cp.wait()
            return carry

        lax.fori_loop(0, nchunk // 4, step, 0)
        pltpu.sync_copy(rows_v, x_hbm.at[pl.ds(w * rpw, rpw), :])

    k = pl.kernel(
        body,
        out_type=jax.ShapeDtypeStruct((B * SLOTS, EMB), jnp.float32),
        mesh=mesh,
        compiler_params=pltpu.CompilerParams(use_tc_tiling_on_sc=False),
        scratch_types=[
            pltpu.VMEM((nchunk, 128), jnp.int32),
            pltpu.VMEM((rpw, EMB), jnp.float32),
            pltpu.SemaphoreType.DMA,
        ],
    )
    return k(table, idx3)


def _dcn_tc(x4d, xraw, cwe, cwd, betag, w1e, w1d, b1, W2, b2, W3, b3, woh, bo):
    nck = DP // 128

    def body(x_ref, xd_ref, cwe_ref, cwd_ref, bg_ref, w1e_ref, w1d_ref,
             b1_ref, w2_ref, b2_ref, w3_ref, b3_ref, woh_ref, bo_ref, out_ref):
        xe = jnp.concatenate(
            [x_ref[:, c].reshape(BM, 128) for c in range(nck)],
            axis=1).astype(jnp.bfloat16)
        xd = xd_ref[...].astype(jnp.bfloat16)
        P = (jnp.dot(xe, cwe_ref[...], preferred_element_type=jnp.float32)
             + jnp.dot(xd, cwd_ref[...], preferred_element_type=jnp.float32))
        c_l = jnp.ones((BM, 1), jnp.float32)
        for l in range(NCROSS):
            c_l = c_l + c_l * P[:, l:l + 1] + bg_ref[0, l]
        gamma = bg_ref[0, NCROSS]
        h = (jnp.dot(xe, w1e_ref[...], preferred_element_type=jnp.float32)
             + jnp.dot(xd, w1d_ref[...], preferred_element_type=jnp.float32))
        h = jnp.maximum(h + b1_ref[...], 0.0).astype(jnp.bfloat16)
        h = jnp.maximum(
            jnp.dot(h, w2_ref[...], preferred_element_type=jnp.float32)
            + b2_ref[...], 0.0).astype(jnp.bfloat16)
        h = jnp.maximum(
            jnp.dot(h, w3_ref[...], preferred_element_type=jnp.float32)
            + b3_ref[...], 0.0).astype(jnp.bfloat16)
        logit = (c_l * P[:, NCROSS:NCROSS + 1] + gamma
                 + jnp.dot(h, woh_ref[...], preferred_element_type=jnp.float32)
                 + bo_ref[0, 0])
        out_ref[...] = (1.0 / (1.0 + jnp.exp(-logit)))[:, 0]

    def wspec(shape):
        return pl.BlockSpec(shape, lambda i: (0,) * len(shape))

    return pl.pallas_call(
        body,
        grid=(B // BM,),
        in_specs=[
            pl.BlockSpec((BM // 8, nck, 8, 128), lambda i: (i, 0, 0, 0)),
            pl.BlockSpec((BM, NDP), lambda i: (i, 0)),
            wspec((DP, NCROSS + 1)),
            wspec((NDP, NCROSS + 1)),
            wspec((1, 8)),
            wspec((DP, H1)),
            wspec((NDP, H1)),
            wspec((1, H1)),
            wspec((H1, H2)),
            wspec((1, H2)),
            wspec((H2, OUT_DIM)),
            wspec((1, OUT_DIM)),
            wspec((OUT_DIM, 1)),
            wspec((1, 1)),
        ],
        out_specs=pl.BlockSpec((BM,), lambda i: (i,)),
        out_shape=jax.ShapeDtypeStruct((B,), jnp.float32),
        compiler_params=pltpu.CompilerParams(
            dimension_semantics=("arbitrary",)),
    )(x4d, xraw, cwe, cwd, betag, w1e, w1d, b1, W2, b2, W3, b3, woh, bo)


def kernel(inputs, embed_tables, cross_w, cross_b, W1, b1, W2, b2, W3, b3, Wo, bo):
    bf16 = jnp.bfloat16
    f32 = jnp.float32
    info = plsc.get_sparse_core_info()
    nc, ns = info.num_cores, info.num_subcores
    nw = nc * ns
    rpw = (B * SLOTS) // nw
    nchunk = rpw // 128
    bpw = B // nw

    sp_idx = inputs[:, ND:].astype(jnp.int32)
    table = embed_tables.reshape(NF * VOCAB, EMB)

    emb_idx = sp_idx + (jnp.arange(NF, dtype=jnp.int32) * VOCAB)[None, :]
    idx_all = jnp.concatenate(
        [emb_idx, emb_idx[:, :SLOTS - NF]], axis=1)
    idx3 = (idx_all.reshape(nw, bpw // 8, 8, SLOTS // 4, 4)
            .transpose(0, 1, 3, 2, 4).reshape(nw, nchunk, 128))

    x_flat = _gather_x(table, idx3, nw, nc, nchunk, rpw)
    x4d = x_flat.reshape(B // 8, DP // 128, 8, 128)

    def embw(m):
        k = m.shape[0]
        return (jnp.concatenate([m, jnp.zeros((k, DP - DE), f32)], axis=1)
                .T.astype(bf16))

    def densew(m):
        k = m.shape[0]
        return (jnp.concatenate([m, jnp.zeros((k, NDP - ND), f32)], axis=1)
                .T.astype(bf16))

    cw5 = jnp.concatenate([cross_w, Wo[:D].reshape(1, D)], axis=0)
    cwe = embw(cw5[:, ND:])
    cwd = densew(cw5[:, :ND])
    w1e = jnp.concatenate(
        [W1[ND:], jnp.zeros((DP - DE, H1), f32)], axis=0).astype(bf16)
    w1d = jnp.concatenate(
        [W1[:ND], jnp.zeros((NDP - ND, H1), f32)], axis=0).astype(bf16)

    bcum = jnp.cumsum(cross_b, axis=0)
    beta = jnp.concatenate([
        jnp.zeros((1,), f32),
        jnp.sum(bcum[:NCROSS - 1] * cross_w[1:], axis=1)])
    gamma = jnp.sum(bcum[NCROSS - 1] * Wo[:D, 0])
    betag = jnp.concatenate(
        [beta, gamma[None], jnp.zeros((3,), f32)]).reshape(1, 8)

    woh = Wo[D:].astype(bf16)

    out = _dcn_tc(
        x4d, inputs, cwe, cwd, betag, w1e, w1d,
        b1.reshape(1, H1), W2.astype(bf16), b2.reshape(1, H2),
        W3.astype(bf16), b3.reshape(1, OUT_DIM), woh, bo.reshape(1, 1))
    return out.reshape(B, 1)


# speedup vs baseline: 2.2987x; 1.0100x over previous
import jax
import jax.numpy as jnp
from jax import lax
from jax.experimental import pallas as pl
from jax.experimental.pallas import tpu as pltpu
from jax.experimental.pallas import tpu_sc as plsc

B = 4096
ND = 13
NF = 26
VOCAB = 1000
EMB = 32
SLOTS = NF + 2
DP = SLOTS * EMB
DE = NF * EMB
D = ND + DE
H1 = 1024
H2 = 1024
OUT_DIM = 256
NCROSS = 4
BM = 1024
NDP = 39


def _gather_x(table, idx2, nw, nc, nchunk, rpw):
    mesh = plsc.VectorSubcoreMesh(core_axis_name="c", subcore_axis_name="s")

    def body(table_hbm, idx_hbm, x_hbm, idx_v, rows_v, sem):
        w = lax.axis_index("s") * nc + lax.axis_index("c")
        pltpu.sync_copy(idx_hbm.at[w], idx_v)

        def step(i, carry):
            cps = [
                pltpu.async_copy(
                    table_hbm.at[idx_v.at[i * 4 + u]],
                    rows_v.at[pl.ds((i * 4 + u) * 128, 128), :], sem,
                )
                for u in range(4)
            ]
            for cp in cps:
                cp.wait()
            return carry

        lax.fori_loop(0, nchunk // 4, step, 0)
        pltpu.sync_copy(rows_v, x_hbm.at[pl.ds(w * rpw, rpw), :])

    k = pl.kernel(
        body,
        out_type=jax.ShapeDtypeStruct((B * SLOTS, EMB), jnp.float32),
        mesh=mesh,
        compiler_params=pltpu.CompilerParams(use_tc_tiling_on_sc=False),
        scratch_types=[
            pltpu.VMEM((nchunk, 128), jnp.int32),
            pltpu.VMEM((rpw, EMB), jnp.float32),
            pltpu.SemaphoreType.DMA,
        ],
    )
    return k(table, idx2)


def _dcn_tc(x4d, xraw, cwe, cwd, betag, w1e, w1d, b1, W2, b2, W3, b3, woh, bo):
    nck = DP // 128

    def body(x_ref, xd_ref, cwe_ref, cwd_ref, bg_ref, w1e_ref, w1d_ref,
             b1_ref, w2_ref, b2_ref, w3_ref, b3_ref, woh_ref, bo_ref, out_ref):
        xe = jnp.concatenate(
            [x_ref[:, c].reshape(BM, 128) for c in range(nck)],
            axis=1).astype(jnp.bfloat16)
        xd = xd_ref[...].astype(jnp.bfloat16)
        P = (jnp.dot(xe, cwe_ref[...], preferred_element_type=jnp.float32)
             + jnp.dot(xd, cwd_ref[...], preferred_element_type=jnp.float32))
        c_l = jnp.ones((BM, 1), jnp.float32)
        for l in range(NCROSS):
            c_l = c_l + c_l * P[:, l:l + 1] + bg_ref[0, l]
        gamma = bg_ref[0, NCROSS]
        h = (jnp.dot(xe, w1e_ref[...], preferred_element_type=jnp.float32)
             + jnp.dot(xd, w1d_ref[...], preferred_element_type=jnp.float32))
        h = jnp.maximum(h + b1_ref[...], 0.0).astype(jnp.bfloat16)
        h = jnp.maximum(
            jnp.dot(h, w2_ref[...], preferred_element_type=jnp.float32)
            + b2_ref[...], 0.0).astype(jnp.bfloat16)
        h = jnp.maximum(
            jnp.dot(h, w3_ref[...], preferred_element_type=jnp.float32)
            + b3_ref[...], 0.0).astype(jnp.bfloat16)
        logit = (c_l * P[:, NCROSS:NCROSS + 1] + gamma
                 + jnp.dot(h, woh_ref[...], preferred_element_type=jnp.float32)
                 + bo_ref[0, 0])
        out_ref[...] = (1.0 / (1.0 + jnp.exp(-logit)))[:, 0]

    def wspec(shape):
        return pl.BlockSpec(shape, lambda i: (0,) * len(shape))

    return pl.pallas_call(
        body,
        grid=(B // BM,),
        in_specs=[
            pl.BlockSpec((BM // 8, nck, 8, 128), lambda i: (i, 0, 0, 0)),
            pl.BlockSpec((BM, NDP), lambda i: (i, 0)),
            wspec((DP, NCROSS + 1)),
            wspec((NDP, NCROSS + 1)),
            wspec((1, 8)),
            wspec((DP, H1)),
            wspec((NDP, H1)),
            wspec((1, H1)),
            wspec((H1, H2)),
            wspec((1, H2)),
            wspec((H2, OUT_DIM)),
            wspec((1, OUT_DIM)),
            wspec((OUT_DIM, 1)),
            wspec((1, 1)),
        ],
        out_specs=pl.BlockSpec((BM,), lambda i: (i,)),
        out_shape=jax.ShapeDtypeStruct((B,), jnp.float32),
        compiler_params=pltpu.CompilerParams(
            dimension_semantics=("arbitrary",)),
    )(x4d, xraw, cwe, cwd, betag, w1e, w1d, b1, W2, b2, W3, b3, woh, bo)


def kernel(inputs, embed_tables, cross_w, cross_b, W1, b1, W2, b2, W3, b3, Wo, bo):
    bf16 = jnp.bfloat16
    f32 = jnp.float32
    info = plsc.get_sparse_core_info()
    nc, ns = info.num_cores, info.num_subcores
    nw = nc * ns
    rpw = (B * SLOTS) // nw
    nchunk = rpw // 128
    bpw = B // nw

    sp_idx = inputs[:, ND:].astype(jnp.int32)
    table = embed_tables.reshape(NF * VOCAB, EMB)

    emb_idx = sp_idx + (jnp.arange(NF, dtype=jnp.int32) * VOCAB)[None, :]
    idx_all = jnp.concatenate(
        [emb_idx, emb_idx[:, :SLOTS - NF]], axis=1)
    idx2 = (idx_all.reshape(nw, bpw // 8, 8, SLOTS // 4, 4)
            .transpose(0, 1, 3, 2, 4).reshape(nw, nchunk, 128))

    x_flat = _gather_x(table, idx2, nw, nc, nchunk, rpw)
    x4d = x_flat.reshape(B // 8, DP // 128, 8, 128)

    def embw(m):
        k = m.shape[0]
        return (jnp.concatenate([m, jnp.zeros((k, DP - DE), f32)], axis=1)
                .T.astype(bf16))

    def densew(m):
        k = m.shape[0]
        return (jnp.concatenate([m, jnp.zeros((k, NDP - ND), f32)], axis=1)
                .T.astype(bf16))

    cw5 = jnp.concatenate([cross_w, Wo[:D].reshape(1, D)], axis=0)
    cwe = embw(cw5[:, ND:])
    cwd = densew(cw5[:, :ND])
    w1e = jnp.concatenate(
        [W1[ND:], jnp.zeros((DP - DE, H1), f32)], axis=0).astype(bf16)
    w1d = jnp.concatenate(
        [W1[:ND], jnp.zeros((NDP - ND, H1), f32)], axis=0).astype(bf16)

    bcum = jnp.cumsum(cross_b, axis=0)
    beta = jnp.concatenate([
        jnp.zeros((1,), f32),
        jnp.sum(bcum[:NCROSS - 1] * cross_w[1:], axis=1)])
    gamma = jnp.sum(bcum[NCROSS - 1] * Wo[:D, 0])
    betag = jnp.concatenate(
        [beta, gamma[None], jnp.zeros((3,), f32)]).reshape(1, 8)

    woh = Wo[D:].astype(bf16)

    out = _dcn_tc(
        x4d, inputs, cwe, cwd, betag, w1e, w1d,
        b1.reshape(1, H1), W2.astype(bf16), b2.reshape(1, H2),
        W3.astype(bf16), b3.reshape(1, OUT_DIM), woh, bo.reshape(1, 1))
    return out.reshape(B, 1)
